# baseline (speedup 1.0000x reference)
import functools

import jax
import jax.numpy as jnp
from jax import lax
from jax.experimental import pallas as pl
from jax.experimental.pallas import tpu as pltpu
from jax.experimental.pallas import tpu_sc as plsc

N_FACTORS = 32
BATCH = 16384
N_ROWS = 1000000
GRANULE = 16
N_GRANULES = N_ROWS // GRANULE

NUM_CORES = 2
NUM_SUBCORES = 16
LANES = 16
NUM_WORKERS = NUM_CORES * NUM_SUBCORES
B_PER_W = BATCH // NUM_WORKERS
CHUNK = 64
N_CHUNKS = B_PER_W // CHUNK
GROUPS_PER_CHUNK = CHUNK // LANES

_mesh = plsc.VectorSubcoreMesh(
    core_axis_name="c", subcore_axis_name="s",
    num_cores=NUM_CORES, num_subcores=NUM_SUBCORES,
)


@functools.partial(
    pl.kernel,
    out_type=jax.ShapeDtypeStruct((BATCH,), jnp.float32),
    mesh=_mesh,
    compiler_params=pltpu.CompilerParams(
        needs_layout_passes=False, use_tc_tiling_on_sc=False),
    scratch_types=dict(
        uidx_v=pltpu.VMEM((B_PER_W,), jnp.int32),
        midx_v=pltpu.VMEM((B_PER_W,), jnp.int32),
        ugran=pltpu.VMEM((CHUNK,), jnp.int32),
        mgran=pltpu.VMEM((CHUNK,), jnp.int32),
        ublk=pltpu.VMEM((N_FACTORS, CHUNK, GRANULE), jnp.float32),
        mblk=pltpu.VMEM((N_FACTORS, CHUNK, GRANULE), jnp.float32),
        out_v=pltpu.VMEM((B_PER_W,), jnp.float32),
        sem=pltpu.SemaphoreType.DMA,
    ),
)
def _sc_body(user_ids, movie_ids, uft3, mft3, out_hbm,
             uidx_v, midx_v, ugran, mgran, ublk, mblk, out_v, sem):
    wid = lax.axis_index("s") * NUM_CORES + lax.axis_index("c")
    base = wid * B_PER_W

    pltpu.sync_copy(user_ids.at[pl.ds(base, B_PER_W)], uidx_v)
    pltpu.sync_copy(movie_ids.at[pl.ds(base, B_PER_W)], midx_v)

    lane = lax.broadcasted_iota(jnp.int32, (LANES,), 0)

    def chunk_body(c, _):
        c0 = c * CHUNK

        def gran_body(g, _):
            sl_src = pl.ds(c0 + g * LANES, LANES)
            sl_dst = pl.ds(g * LANES, LANES)
            ugran[sl_dst] = uidx_v[sl_src] >> 4
            mgran[sl_dst] = midx_v[sl_src] >> 4
            return 0

        lax.fori_loop(0, GROUPS_PER_CHUNK, gran_body, 0)

        for f in range(N_FACTORS):
            pltpu.async_copy(uft3.at[f].at[ugran], ublk.at[f], sem)
            pltpu.async_copy(mft3.at[f].at[mgran], mblk.at[f], sem)

        pltpu.make_async_copy(uft3.at[:, pl.ds(0, CHUNK), :], ublk, sem).wait()
        pltpu.make_async_copy(mft3.at[:, pl.ds(0, CHUNK), :], mblk, sem).wait()

        def dot_body(g, _):
            sl = pl.ds(c0 + g * LANES, LANES)
            ul = uidx_v[sl] & 15
            ml = midx_v[sl] & 15
            jvec = g * LANES + lane
            acc = jnp.zeros((LANES,), jnp.float32)
            for f in range(N_FACTORS):
                fv = jnp.full((LANES,), f, jnp.int32)
                a = plsc.load_gather(ublk, [fv, jvec, ul])
                b = plsc.load_gather(mblk, [fv, jvec, ml])
                acc = acc + a * b
            out_v[sl] = acc
            return 0

        lax.fori_loop(0, GROUPS_PER_CHUNK, dot_body, 0)
        return 0

    lax.fori_loop(0, N_CHUNKS, chunk_body, 0)

    pltpu.sync_copy(out_v, out_hbm.at[pl.ds(base, B_PER_W)])


def kernel(user_ids, movie_ids, user_factors, movie_factors):
    uft3 = user_factors.T.reshape(N_FACTORS, N_GRANULES, GRANULE)
    mft3 = movie_factors.T.reshape(N_FACTORS, N_GRANULES, GRANULE)
    out = _sc_body(
        user_ids.astype(jnp.int32),
        movie_ids.astype(jnp.int32),
        uft3,
        mft3,
    )
    return out.reshape(-1, 1)


# speedup vs baseline: 1.0942x; 1.0942x over previous
import functools

import jax
import jax.numpy as jnp
from jax import lax
from jax.experimental import pallas as pl
from jax.experimental.pallas import tpu as pltpu
from jax.experimental.pallas import tpu_sc as plsc

N_FACTORS = 32
BATCH = 16384
N_ROWS = 1000000
GRANULE = 16
N_GRANULES = N_ROWS // GRANULE

NUM_CORES = 2
NUM_SUBCORES = 16
LANES = 16
NUM_WORKERS = NUM_CORES * NUM_SUBCORES
B_PER_W = BATCH // NUM_WORKERS
CHUNK = 64
N_CHUNKS = B_PER_W // CHUNK
GROUPS_PER_CHUNK = CHUNK // LANES
N_GROUPS = B_PER_W // LANES

_mesh = plsc.VectorSubcoreMesh(
    core_axis_name="c", subcore_axis_name="s",
    num_cores=NUM_CORES, num_subcores=NUM_SUBCORES,
)


@functools.partial(
    pl.kernel,
    out_type=jax.ShapeDtypeStruct((BATCH,), jnp.float32),
    mesh=_mesh,
    compiler_params=pltpu.CompilerParams(
        needs_layout_passes=False, use_tc_tiling_on_sc=False),
    scratch_types=dict(
        uidx_v=pltpu.VMEM((B_PER_W,), jnp.int32),
        midx_v=pltpu.VMEM((B_PER_W,), jnp.int32),
        ugidx=pltpu.VMEM((N_FACTORS, B_PER_W), jnp.int32),
        mgidx=pltpu.VMEM((N_FACTORS, B_PER_W), jnp.int32),
        ublk=pltpu.VMEM((N_FACTORS, CHUNK, GRANULE), jnp.float32),
        mblk=pltpu.VMEM((N_FACTORS, CHUNK, GRANULE), jnp.float32),
        out_v=pltpu.VMEM((B_PER_W,), jnp.float32),
        sem=pltpu.SemaphoreType.DMA,
    ),
)
def _sc_body(user_ids, movie_ids, ufg, mfg, out_hbm,
             uidx_v, midx_v, ugidx, mgidx, ublk, mblk, out_v, sem):
    wid = lax.axis_index("s") * NUM_CORES + lax.axis_index("c")
    base = wid * B_PER_W

    pltpu.sync_copy(user_ids.at[pl.ds(base, B_PER_W)], uidx_v)
    pltpu.sync_copy(movie_ids.at[pl.ds(base, B_PER_W)], midx_v)

    lane = lax.broadcasted_iota(jnp.int32, (LANES,), 0)

    def gidx_body(g, _):
        sl = pl.ds(g * LANES, LANES)
        ug = uidx_v[sl] >> 4
        mg = midx_v[sl] >> 4
        for f in range(N_FACTORS):
            off = jnp.int32(f * N_GRANULES)
            ugidx[f, sl] = ug + off
            mgidx[f, sl] = mg + off
        return 0

    lax.fori_loop(0, N_GROUPS, gidx_body, 0)

    def chunk_body(c, _):
        c0 = c * CHUNK

        copies = []
        for f in range(N_FACTORS):
            copies.append(
                pltpu.async_copy(ufg.at[ugidx.at[f, pl.ds(c0, CHUNK)]],
                                 ublk.at[f], sem))
            copies.append(
                pltpu.async_copy(mfg.at[mgidx.at[f, pl.ds(c0, CHUNK)]],
                                 mblk.at[f], sem))
        for cp in copies:
            cp.wait()

        def dot_body(g, _):
            sl = pl.ds(c0 + g * LANES, LANES)
            ul = uidx_v[sl] & 15
            ml = midx_v[sl] & 15
            jvec = g * LANES + lane
            acc = jnp.zeros((LANES,), jnp.float32)
            for f in range(N_FACTORS):
                fv = jnp.full((LANES,), f, jnp.int32)
                a = plsc.load_gather(ublk, [fv, jvec, ul])
                b = plsc.load_gather(mblk, [fv, jvec, ml])
                acc = acc + a * b
            out_v[sl] = acc
            return 0

        lax.fori_loop(0, GROUPS_PER_CHUNK, dot_body, 0)
        return 0

    lax.fori_loop(0, N_CHUNKS, chunk_body, 0)

    pltpu.sync_copy(out_v, out_hbm.at[pl.ds(base, B_PER_W)])


def kernel(user_ids, movie_ids, user_factors, movie_factors):
    out = _sc_body(
        user_ids.astype(jnp.int32),
        movie_ids.astype(jnp.int32),
        user_factors.T.reshape(N_FACTORS * N_GRANULES, GRANULE),
        movie_factors.T.reshape(N_FACTORS * N_GRANULES, GRANULE),
    )
    return out.reshape(-1, 1)


# speedup vs baseline: 1.0958x; 1.0014x over previous
import functools

import jax
import jax.numpy as jnp
from jax import lax
from jax.experimental import pallas as pl
from jax.experimental.pallas import tpu as pltpu
from jax.experimental.pallas import tpu_sc as plsc

N_FACTORS = 32
BATCH = 16384
N_ROWS = 1000000
GRANULE = 16
N_GRANULES = N_ROWS // GRANULE

NUM_CORES = 2
NUM_SUBCORES = 16
LANES = 16
NUM_WORKERS = NUM_CORES * NUM_SUBCORES
B_PER_W = BATCH // NUM_WORKERS
CHUNK = 128
N_CHUNKS = B_PER_W // CHUNK
GROUPS_PER_CHUNK = CHUNK // LANES
N_GROUPS = B_PER_W // LANES
F_HALF = N_FACTORS // 2

_mesh = plsc.VectorSubcoreMesh(
    core_axis_name="c", subcore_axis_name="s",
    num_cores=NUM_CORES, num_subcores=NUM_SUBCORES,
)


@functools.partial(
    pl.kernel,
    out_type=jax.ShapeDtypeStruct((BATCH,), jnp.float32),
    mesh=_mesh,
    compiler_params=pltpu.CompilerParams(
        needs_layout_passes=False, use_tc_tiling_on_sc=False),
    scratch_types=dict(
        uidx_v=pltpu.VMEM((B_PER_W,), jnp.int32),
        midx_v=pltpu.VMEM((B_PER_W,), jnp.int32),
        ugidx=pltpu.VMEM((N_FACTORS, N_CHUNKS, CHUNK), jnp.int32),
        mgidx=pltpu.VMEM((N_FACTORS, N_CHUNKS, CHUNK), jnp.int32),
        ublk=pltpu.VMEM((F_HALF, CHUNK, GRANULE), jnp.float32),
        mblk=pltpu.VMEM((F_HALF, CHUNK, GRANULE), jnp.float32),
        out_v=pltpu.VMEM((B_PER_W,), jnp.float32),
        sem=pltpu.SemaphoreType.DMA,
    ),
)
def _sc_body(user_ids, movie_ids, ufg, mfg, out_hbm,
             uidx_v, midx_v, ugidx, mgidx, ublk, mblk, out_v, sem):
    wid = lax.axis_index("s") * NUM_CORES + lax.axis_index("c")
    base = wid * B_PER_W

    pltpu.sync_copy(user_ids.at[pl.ds(base, B_PER_W)], uidx_v)
    pltpu.sync_copy(movie_ids.at[pl.ds(base, B_PER_W)], midx_v)

    lane = lax.broadcasted_iota(jnp.int32, (LANES,), 0)

    def gidx_body(g, _):
        c = g // GROUPS_PER_CHUNK
        r = g % GROUPS_PER_CHUNK
        sl_src = pl.ds(g * LANES, LANES)
        sl_dst = pl.ds(r * LANES, LANES)
        ug = uidx_v[sl_src] >> 4
        mg = midx_v[sl_src] >> 4
        for f in range(N_FACTORS):
            off = jnp.int32(f * N_GRANULES)
            ugidx[f, c, sl_dst] = ug + off
            mgidx[f, c, sl_dst] = mg + off
        return 0

    lax.fori_loop(0, N_GROUPS, gidx_body, 0)

    def chunk_body(c, _):
        c0 = c * CHUNK

        for h in range(2):
            copies = []
            for j in range(F_HALF):
                f = h * F_HALF + j
                copies.append(
                    pltpu.async_copy(ufg.at[ugidx.at[f, c]], ublk.at[j], sem))
                copies.append(
                    pltpu.async_copy(mfg.at[mgidx.at[f, c]], mblk.at[j], sem))
            for cp in copies:
                cp.wait()

            def dot_body(g, acc_unused, h=h):
                sl = pl.ds(c0 + g * LANES, LANES)
                ul = uidx_v[sl] & 15
                ml = midx_v[sl] & 15
                jvec = g * LANES + lane
                acc = jnp.zeros((LANES,), jnp.float32)
                for j in range(F_HALF):
                    fv = jnp.full((LANES,), j, jnp.int32)
                    a = plsc.load_gather(ublk, [fv, jvec, ul])
                    b = plsc.load_gather(mblk, [fv, jvec, ml])
                    acc = acc + a * b
                if h == 0:
                    out_v[sl] = acc
                else:
                    out_v[sl] = out_v[sl] + acc
                return 0

            lax.fori_loop(0, GROUPS_PER_CHUNK, dot_body, 0)
        return 0

    lax.fori_loop(0, N_CHUNKS, chunk_body, 0)

    pltpu.sync_copy(out_v, out_hbm.at[pl.ds(base, B_PER_W)])


def kernel(user_ids, movie_ids, user_factors, movie_factors):
    out = _sc_body(
        user_ids.astype(jnp.int32),
        movie_ids.astype(jnp.int32),
        user_factors.T.reshape(N_FACTORS * N_GRANULES, GRANULE),
        movie_factors.T.reshape(N_FACTORS * N_GRANULES, GRANULE),
    )
    return out.reshape(-1, 1)
